# Initial kernel scaffold; baseline (speedup 1.0000x reference)
import jax
import jax.numpy as jnp
from jax.experimental import pallas as pl


def kernel(inputs, exp, xyzstorays, embeddings_mean, embeddings):
    raise NotImplementedError("write your pallas kernel here")


# speedup vs baseline: 9.2108x; 9.2108x over previous
import functools

import numpy as np
import jax
import jax.numpy as jnp
from jax import lax
from jax.experimental import pallas as pl
from jax.experimental.pallas import tpu as pltpu
from jax.experimental.pallas import tpu_sc as plsc

_INPUT_DIM = 3
_NUM_LEVELS = 16
_LEVEL_DIM = 2
_BASE_RES = 16
_LOG2_HASH = 17
_BASIS_NUM = 8
_N_FRAMES = 8
_N_POINTS = 131072

_LEVEL_PARAMS = []
_off = 0
for _l in range(_NUM_LEVELS):
    _res = int(np.ceil(_BASE_RES * 2.0 ** _l))
    _params = min(2 ** _LOG2_HASH, (_res + 1) ** _INPUT_DIM)
    _scale = float(np.exp2(float(_l)) * _BASE_RES - 1.0)
    _resolution = int(np.ceil(_scale)) + 1
    _use_hash = (_resolution + 1) ** _INPUT_DIM > _params
    _LEVEL_PARAMS.append(dict(scale=_scale, res=_resolution, hashmap=_params,
                              offset=_off, use_hash=_use_hash))
    _off += _params
_TOTAL = _off

_P1 = int(np.uint32(2654435761).astype(np.int32))
_P2 = int(np.uint32(805459861).astype(np.int32))

_NW = 32
_PPW = _N_POINTS // _NW
_NG = _PPW // 16
_OUT_D = _NUM_LEVELS * _LEVEL_DIM
_MAX_HM = 2 ** _LOG2_HASH


def _align8(n):
    return (n + 7) // 8 * 8


def _combine_body(exp_ref, me_ref, mo_ref, ee_ref, eo_ref, out_ref):
    e = exp_ref[...]
    b_even = jnp.concatenate([me_ref[...], ee_ref[...]], axis=0)
    b_odd = jnp.concatenate([mo_ref[...], eo_ref[...]], axis=0)
    dn = (((1,), (0,)), ((), ()))
    c0 = lax.dot_general(e, b_even, dn, preferred_element_type=jnp.float32)
    c1 = lax.dot_general(e, b_odd, dn, preferred_element_type=jnp.float32)
    u0 = lax.bitcast_convert_type(c0.astype(jnp.bfloat16), jnp.uint16)
    u1 = lax.bitcast_convert_type(c1.astype(jnp.bfloat16), jnp.uint16)
    word = u0.astype(jnp.uint32) | (u1.astype(jnp.uint32) << 16)
    out_ref[...] = lax.bitcast_convert_type(word, jnp.int32)


def _combine_level(exp, me, mo, ee, eo, hm8):
    bn = min(hm8, 65536)
    grid = hm8 // bn if hm8 % bn == 0 else (hm8 + bn - 1) // bn
    return pl.pallas_call(
        _combine_body,
        grid=(grid,),
        in_specs=[
            pl.BlockSpec((_BASIS_NUM, _BASIS_NUM), lambda i: (0, 0)),
            pl.BlockSpec((1, bn), lambda i: (0, i)),
            pl.BlockSpec((1, bn), lambda i: (0, i)),
            pl.BlockSpec((_BASIS_NUM - 1, bn), lambda i: (0, i)),
            pl.BlockSpec((_BASIS_NUM - 1, bn), lambda i: (0, i)),
        ],
        out_specs=pl.BlockSpec((_BASIS_NUM, bn), lambda i: (0, i)),
        out_shape=jax.ShapeDtypeStruct((_BASIS_NUM, hm8), jnp.int32),
    )(exp, me, mo, ee, eo)


def _sc_body(*args):
    tbls = args[:_NUM_LEVELS]
    (xs_hbm, ys_hbm, zs_hbm, rays_hbm, out_hbm,
     xs, ys, zs, fr, idx_buf, gb, ob0, ob1, sem) = args[_NUM_LEVELS:]
    cid = lax.axis_index("c")
    sid = lax.axis_index("s")
    wid = sid * 2 + cid
    lanes = jnp.arange(16, dtype=jnp.int32)
    wbase = wid * _PPW

    pltpu.sync_copy(xs_hbm.at[pl.ds(wbase, _PPW)], xs)
    pltpu.sync_copy(ys_hbm.at[pl.ds(wbase, _PPW)], ys)
    pltpu.sync_copy(zs_hbm.at[pl.ds(wbase, _PPW)], zs)
    pltpu.sync_copy(rays_hbm.at[pl.ds(wbase, _PPW)], fr)

    def fr_body(g, _):
        sl = pl.ds(g * 16, 16)
        fr[sl] = fr[sl] >> 10
        return ()
    lax.fori_loop(0, _NG, fr_body, (), unroll=False)

    for lvl in range(_NUM_LEVELS):
        p = _LEVEL_PARAMS[lvl]
        scale = jnp.float32(p["scale"])
        hm = p["hashmap"]
        hm8 = _align8(hm)
        tbl = tbls[lvl]

        def corner_setup(g, p=p, scale=scale, hm=hm):
            sl = pl.ds(g * 16, 16)
            px = xs[sl] * scale + 0.5
            py = ys[sl] * scale + 0.5
            pz = zs[sl] * scale + 0.5
            ix = px.astype(jnp.int32)
            iy = py.astype(jnp.int32)
            iz = pz.astype(jnp.int32)
            fx = px - ix.astype(jnp.float32)
            fy = py - iy.astype(jnp.float32)
            fz = pz - iz.astype(jnp.float32)
            if p["use_hash"]:
                ya = iy * _P1
                za = iz * _P2
                mask = hm - 1
                def cidx(cx, cy, cz):
                    return (((ix + cx)
                             ^ (ya + cy * _P1)
                             ^ (za + cz * _P2)) & mask)
            else:
                r1 = p["res"] + 1
                ya = iy * r1
                za = iz * (r1 * r1)
                def cidx(cx, cy, cz):
                    return (ix + cx) + (ya + cy * r1) + (za + cz * (r1 * r1))
            return cidx, (fx, fy, fz)

        def pass_a(g, _, p=p, scale=scale, hm=hm, hm8=hm8):
            cidx, _fracs = corner_setup(g, p=p, scale=scale, hm=hm)
            fbv = fr[pl.ds(g * 16, 16)] * hm8
            gbase = g * 128
            for c in range(8):
                cx, cy, cz = c & 1, (c >> 1) & 1, (c >> 2) & 1
                idx_buf[pl.ds(gbase + c * 16, 16)] = cidx(cx, cy, cz) + fbv
            return ()

        lax.fori_loop(0, _NG, pass_a, (), unroll=False)
        pltpu.async_copy(tbl.at[idx_buf], gb, sem).wait()

        def pass_b(g, _, p=p, scale=scale, hm=hm):
            _cidx, (fx, fy, fz) = corner_setup(g, p=p, scale=scale, hm=hm)
            wx0, wy0, wz0 = 1.0 - fx, 1.0 - fy, 1.0 - fz
            wxy = [wx0 * wy0, fx * wy0, wx0 * fy, fx * fy]
            gbase = g * 128
            acc0 = jnp.zeros((16,), jnp.float32)
            acc1 = jnp.zeros((16,), jnp.float32)
            for c in range(8):
                cx, cy, cz = c & 1, (c >> 1) & 1, (c >> 2) & 1
                w = wxy[cx + 2 * cy] * (fz if cz else wz0)
                vi = plsc.load_gather(gb, [gbase + c * 16 + lanes])
                v0 = plsc.bitcast(vi << 16, jnp.float32)
                v1 = plsc.bitcast(vi & jnp.int32(-65536), jnp.float32)
                acc0 = acc0 + w * v0
                acc1 = acc1 + w * v1
            sl = pl.ds(g * 16, 16)
            ob0[sl] = acc0
            ob1[sl] = acc1
            return ()

        lax.fori_loop(0, _NG, pass_b, (), unroll=False)

        pltpu.sync_copy(ob0, out_hbm.at[2 * lvl, pl.ds(wbase, _PPW)])
        pltpu.sync_copy(ob1, out_hbm.at[2 * lvl + 1, pl.ds(wbase, _PPW)])


def _sc_encode(tbls, xs, ys, zs, rays):
    mesh = plsc.VectorSubcoreMesh(core_axis_name="c", subcore_axis_name="s",
                                  num_cores=2, num_subcores=16)
    f = functools.partial(
        pl.kernel,
        out_type=jax.ShapeDtypeStruct((_OUT_D, _N_POINTS), jnp.float32),
        mesh=mesh,
        compiler_params=pltpu.CompilerParams(needs_layout_passes=False),
        scratch_types=[
            pltpu.VMEM((_PPW,), jnp.float32),
            pltpu.VMEM((_PPW,), jnp.float32),
            pltpu.VMEM((_PPW,), jnp.float32),
            pltpu.VMEM((_PPW,), jnp.int32),
            pltpu.VMEM((_PPW * 8,), jnp.int32),
            pltpu.VMEM((_PPW * 8,), jnp.int32),
            pltpu.VMEM((_PPW,), jnp.float32),
            pltpu.VMEM((_PPW,), jnp.float32),
            pltpu.SemaphoreType.DMA,
        ],
    )(_sc_body)
    return f(*tbls, xs, ys, zs, rays)


def kernel(inputs, exp, xyzstorays, embeddings_mean, embeddings):
    me = embeddings_mean[:, :, 0]
    mo = embeddings_mean[:, :, 1]
    ee = embeddings[:, :, 0]
    eo = embeddings[:, :, 1]
    tbls = []
    for p in _LEVEL_PARAMS:
        off, hm = p["offset"], p["hashmap"]
        hm8 = _align8(hm)
        pad = hm8 - hm
        def cut(a, off=off, hm=hm, pad=pad):
            sl = a[:, off:off + hm]
            if pad:
                sl = jnp.pad(sl, ((0, 0), (0, pad)))
            return sl
        tbls.append(_combine_level(exp, cut(me), cut(mo), cut(ee), cut(eo),
                                   hm8).reshape(-1))
    xs = inputs[:, 0]
    ys = inputs[:, 1]
    zs = inputs[:, 2]
    out = _sc_encode(tbls, xs, ys, zs, xyzstorays.astype(jnp.int32))
    return out.T


# speedup vs baseline: 11.1719x; 1.2129x over previous
import functools

import numpy as np
import jax
import jax.numpy as jnp
from jax import lax
from jax.experimental import pallas as pl
from jax.experimental.pallas import tpu as pltpu
from jax.experimental.pallas import tpu_sc as plsc

_INPUT_DIM = 3
_NUM_LEVELS = 16
_LEVEL_DIM = 2
_BASE_RES = 16
_LOG2_HASH = 17
_BASIS_NUM = 8
_N_FRAMES = 8
_N_POINTS = 131072

_LEVEL_PARAMS = []
_off = 0
for _l in range(_NUM_LEVELS):
    _res = int(np.ceil(_BASE_RES * 2.0 ** _l))
    _params = min(2 ** _LOG2_HASH, (_res + 1) ** _INPUT_DIM)
    _scale = float(np.exp2(float(_l)) * _BASE_RES - 1.0)
    _resolution = int(np.ceil(_scale)) + 1
    _use_hash = (_resolution + 1) ** _INPUT_DIM > _params
    _LEVEL_PARAMS.append(dict(scale=_scale, res=_resolution, hashmap=_params,
                              offset=_off, use_hash=_use_hash))
    _off += _params
_TOTAL = _off

_P1 = int(np.uint32(2654435761).astype(np.int32))
_P2 = int(np.uint32(805459861).astype(np.int32))

_NW = 32
_PPW = _N_POINTS // _NW
_NG = _PPW // 16
_OUT_D = _NUM_LEVELS * _LEVEL_DIM
_MAX_HM = 2 ** _LOG2_HASH
_CB = 4096
_CP = 2048


def _align8(n):
    return (n + 7) // 8 * 8


def _combine_body(exp_ref, me_ref, mo_ref, ee_ref, eo_ref, out_ref):
    e = exp_ref[...]
    b_even = jnp.concatenate([me_ref[...], ee_ref[...]], axis=0)
    b_odd = jnp.concatenate([mo_ref[...], eo_ref[...]], axis=0)
    dn = (((1,), (0,)), ((), ()))
    c0 = lax.dot_general(e, b_even, dn, preferred_element_type=jnp.float32)
    c1 = lax.dot_general(e, b_odd, dn, preferred_element_type=jnp.float32)
    u0 = lax.bitcast_convert_type(c0.astype(jnp.bfloat16), jnp.uint16)
    u1 = lax.bitcast_convert_type(c1.astype(jnp.bfloat16), jnp.uint16)
    word = u0.astype(jnp.uint32) | (u1.astype(jnp.uint32) << 16)
    out_ref[...] = lax.bitcast_convert_type(word, jnp.int32)


def _combine_level(exp, me, mo, ee, eo, hm8):
    bn = min(hm8, 65536)
    grid = hm8 // bn if hm8 % bn == 0 else (hm8 + bn - 1) // bn
    return pl.pallas_call(
        _combine_body,
        grid=(grid,),
        in_specs=[
            pl.BlockSpec((_BASIS_NUM, _BASIS_NUM), lambda i: (0, 0)),
            pl.BlockSpec((1, bn), lambda i: (0, i)),
            pl.BlockSpec((1, bn), lambda i: (0, i)),
            pl.BlockSpec((_BASIS_NUM - 1, bn), lambda i: (0, i)),
            pl.BlockSpec((_BASIS_NUM - 1, bn), lambda i: (0, i)),
        ],
        out_specs=pl.BlockSpec((_BASIS_NUM, bn), lambda i: (0, i)),
        out_shape=jax.ShapeDtypeStruct((_BASIS_NUM, hm8), jnp.int32),
    )(exp, me, mo, ee, eo)


def _sc_body(*args):
    tbls = args[:_NUM_LEVELS]
    (xs_hbm, ys_hbm, zs_hbm, rays_hbm, out_hbm,
     xs, ys, zs, fr, idx_buf, gb, ob0, ob1, bounce, spm, sem) = args[_NUM_LEVELS:]
    cid = lax.axis_index("c")
    sid = lax.axis_index("s")
    wid = sid * 2 + cid
    lanes = jnp.arange(16, dtype=jnp.int32)
    wbase = wid * _PPW

    pltpu.sync_copy(xs_hbm.at[pl.ds(wbase, _PPW)], xs)
    pltpu.sync_copy(ys_hbm.at[pl.ds(wbase, _PPW)], ys)
    pltpu.sync_copy(zs_hbm.at[pl.ds(wbase, _PPW)], zs)
    pltpu.sync_copy(rays_hbm.at[pl.ds(wbase, _PPW)], fr)

    def fr_body(g, _):
        sl = pl.ds(g * 16, 16)
        fr[sl] = fr[sl] >> 10
        return ()
    lax.fori_loop(0, _NG, fr_body, (), unroll=False)

    for lvl in range(_NUM_LEVELS):
        p = _LEVEL_PARAMS[lvl]
        scale = jnp.float32(p["scale"])
        hm = p["hashmap"]
        hm8 = _align8(hm)
        tbl = tbls[lvl]

        w_total = hm8 * _N_FRAMES
        nchunks = (w_total + _CB - 1) // _CB
        rounds = (nchunks + 15) // 16

        def stage_round(j, _):
            k = sid + 16 * j
            coff = k * _CB

            @pl.when(coff < w_total)
            def _():
                clen = jnp.minimum(_CB, w_total - coff)
                pltpu.sync_copy(tbl.at[pl.ds(coff, _CB)], bounce)
                pltpu.sync_copy(bounce, spm.at[pl.ds(coff, _CB)])
            return ()

        if w_total % _CB == 0:
            lax.fori_loop(0, rounds, stage_round, (), unroll=False)
        else:
            for k in range(nchunks):
                coff = k * _CB
                clen = min(_CB, w_total - coff)

                @pl.when(sid == (k % 16))
                def _(coff=coff, clen=clen):
                    pltpu.sync_copy(tbl.at[pl.ds(coff, clen)],
                                    bounce.at[pl.ds(0, clen)])
                    pltpu.sync_copy(bounce.at[pl.ds(0, clen)],
                                    spm.at[pl.ds(coff, clen)])
        plsc.subcore_barrier()

        def corner_setup(g, p=p, scale=scale, hm=hm):
            sl = pl.ds(g * 16, 16)
            px = xs[sl] * scale + 0.5
            py = ys[sl] * scale + 0.5
            pz = zs[sl] * scale + 0.5
            ix = px.astype(jnp.int32)
            iy = py.astype(jnp.int32)
            iz = pz.astype(jnp.int32)
            fx = px - ix.astype(jnp.float32)
            fy = py - iy.astype(jnp.float32)
            fz = pz - iz.astype(jnp.float32)
            if p["use_hash"]:
                ya = iy * _P1
                za = iz * _P2
                mask = hm - 1
                def cidx(cx, cy, cz):
                    return (((ix + cx)
                             ^ (ya + cy * _P1)
                             ^ (za + cz * _P2)) & mask)
            else:
                r1 = p["res"] + 1
                ya = iy * r1
                za = iz * (r1 * r1)
                def cidx(cx, cy, cz):
                    return (ix + cx) + (ya + cy * r1) + (za + cz * (r1 * r1))
            return cidx, (fx, fy, fz)

        def pass_a(g, _, p=p, scale=scale, hm=hm, hm8=hm8):
            cidx, _fracs = corner_setup(g, p=p, scale=scale, hm=hm)
            fbv = fr[pl.ds(g * 16, 16)] * hm8
            gbase = (g % (_CP // 16)) * 128
            for c in range(8):
                cx, cy, cz = c & 1, (c >> 1) & 1, (c >> 2) & 1
                idx_buf[pl.ds(gbase + c * 16, 16)] = cidx(cx, cy, cz) + fbv
            return ()

        def pass_b(g, _, p=p, scale=scale, hm=hm):
            _cidx, (fx, fy, fz) = corner_setup(g, p=p, scale=scale, hm=hm)
            wx0, wy0, wz0 = 1.0 - fx, 1.0 - fy, 1.0 - fz
            wxy = [wx0 * wy0, fx * wy0, wx0 * fy, fx * fy]
            gbase = (g % (_CP // 16)) * 128
            acc0 = jnp.zeros((16,), jnp.float32)
            acc1 = jnp.zeros((16,), jnp.float32)
            for c in range(8):
                cx, cy, cz = c & 1, (c >> 1) & 1, (c >> 2) & 1
                w = wxy[cx + 2 * cy] * (fz if cz else wz0)
                vi = plsc.load_gather(gb, [gbase + c * 16 + lanes])
                v0 = plsc.bitcast(vi << 16, jnp.float32)
                v1 = plsc.bitcast(vi & jnp.int32(-65536), jnp.float32)
                acc0 = acc0 + w * v0
                acc1 = acc1 + w * v1
            sl = pl.ds(g * 16, 16)
            ob0[sl] = acc0
            ob1[sl] = acc1
            return ()

        for half in range(_PPW // _CP):
            g0 = half * (_CP // 16)
            lax.fori_loop(g0, g0 + _CP // 16, pass_a, (), unroll=False)
            pltpu.async_copy(spm.at[idx_buf], gb, sem).wait()
            lax.fori_loop(g0, g0 + _CP // 16, pass_b, (), unroll=False)

        pltpu.sync_copy(ob0, out_hbm.at[2 * lvl, pl.ds(wbase, _PPW)])
        pltpu.sync_copy(ob1, out_hbm.at[2 * lvl + 1, pl.ds(wbase, _PPW)])
        plsc.subcore_barrier()


def _sc_encode(tbls, xs, ys, zs, rays):
    mesh = plsc.VectorSubcoreMesh(core_axis_name="c", subcore_axis_name="s",
                                  num_cores=2, num_subcores=16)
    f = functools.partial(
        pl.kernel,
        out_type=jax.ShapeDtypeStruct((_OUT_D, _N_POINTS), jnp.float32),
        mesh=mesh,
        compiler_params=pltpu.CompilerParams(needs_layout_passes=False),
        scratch_types=[
            pltpu.VMEM((_PPW,), jnp.float32),
            pltpu.VMEM((_PPW,), jnp.float32),
            pltpu.VMEM((_PPW,), jnp.float32),
            pltpu.VMEM((_PPW,), jnp.int32),
            pltpu.VMEM((_CP * 8,), jnp.int32),
            pltpu.VMEM((_CP * 8,), jnp.int32),
            pltpu.VMEM((_PPW,), jnp.float32),
            pltpu.VMEM((_PPW,), jnp.float32),
            pltpu.VMEM((_CB,), jnp.int32),
            pltpu.VMEM_SHARED((_N_FRAMES * _MAX_HM,), jnp.int32),
            pltpu.SemaphoreType.DMA,
        ],
    )(_sc_body)
    return f(*tbls, xs, ys, zs, rays)


def kernel(inputs, exp, xyzstorays, embeddings_mean, embeddings):
    me = embeddings_mean[:, :, 0]
    mo = embeddings_mean[:, :, 1]
    ee = embeddings[:, :, 0]
    eo = embeddings[:, :, 1]
    tbls = []
    for p in _LEVEL_PARAMS:
        off, hm = p["offset"], p["hashmap"]
        hm8 = _align8(hm)
        pad = hm8 - hm
        def cut(a, off=off, hm=hm, pad=pad):
            sl = a[:, off:off + hm]
            if pad:
                sl = jnp.pad(sl, ((0, 0), (0, pad)))
            return sl
        tbls.append(_combine_level(exp, cut(me), cut(mo), cut(ee), cut(eo),
                                   hm8).reshape(-1))
    xs = inputs[:, 0]
    ys = inputs[:, 1]
    zs = inputs[:, 2]
    out = _sc_encode(tbls, xs, ys, zs, xyzstorays.astype(jnp.int32))
    return out.T


# speedup vs baseline: 14.9909x; 1.3418x over previous
import functools

import numpy as np
import jax
import jax.numpy as jnp
from jax import lax
from jax.experimental import pallas as pl
from jax.experimental.pallas import tpu as pltpu
from jax.experimental.pallas import tpu_sc as plsc

_INPUT_DIM = 3
_NUM_LEVELS = 16
_LEVEL_DIM = 2
_BASE_RES = 16
_LOG2_HASH = 17
_BASIS_NUM = 8
_N_FRAMES = 8
_N_POINTS = 131072

_LEVEL_PARAMS = []
_off = 0
for _l in range(_NUM_LEVELS):
    _res = int(np.ceil(_BASE_RES * 2.0 ** _l))
    _params = min(2 ** _LOG2_HASH, (_res + 1) ** _INPUT_DIM)
    _scale = float(np.exp2(float(_l)) * _BASE_RES - 1.0)
    _resolution = int(np.ceil(_scale)) + 1
    _use_hash = (_resolution + 1) ** _INPUT_DIM > _params
    _LEVEL_PARAMS.append(dict(scale=_scale, res=_resolution, hashmap=_params,
                              offset=_off, use_hash=_use_hash))
    _off += _params
_TOTAL = _off

_P1 = int(np.uint32(2654435761).astype(np.int32))
_P2 = int(np.uint32(805459861).astype(np.int32))

_NW = 32
_PPW = _N_POINTS // _NW
_NG = _PPW // 16
_OUT_D = _NUM_LEVELS * _LEVEL_DIM
_HM_BIG = 2 ** _LOG2_HASH
_BIG0 = 2
_NBIG = _NUM_LEVELS - _BIG0
_WBIG = _N_FRAMES * _HM_BIG
_CB = 4096
_TW = 29 * 65536
_LVL_STAGE = []
for _p in _LEVEL_PARAMS:
    _o = _p["offset"]
    _pad0 = _o & 7
    _stride = (_pad0 + _p["hashmap"] + _CB - 1) // _CB * _CB
    _LVL_STAGE.append(dict(off8=_o & ~7, pad0=_pad0, stride=_stride))
_CP = 1024
_CW = _CP * 8
_NCH = _PPW // _CP


def _pad1024(n):
    return (n + 1023) // 1024 * 1024


def _combine_body(exp_ref, me_ref, mo_ref, ee_ref, eo_ref, out_ref):
    e = exp_ref[...]
    b_even = jnp.concatenate([me_ref[...], ee_ref[...]], axis=0)
    b_odd = jnp.concatenate([mo_ref[...], eo_ref[...]], axis=0)
    dn = (((1,), (0,)), ((), ()))
    c0 = lax.dot_general(e, b_even, dn, preferred_element_type=jnp.float32)
    c1 = lax.dot_general(e, b_odd, dn, preferred_element_type=jnp.float32)
    u0 = lax.bitcast_convert_type(c0.astype(jnp.bfloat16), jnp.uint16)
    u1 = lax.bitcast_convert_type(c1.astype(jnp.bfloat16), jnp.uint16)
    word = u0.astype(jnp.uint32) | (u1.astype(jnp.uint32) << 16)
    out_ref[...] = lax.bitcast_convert_type(word, jnp.int32)


def _combine_full(exp, me, mo, ee, eo):
    bn = 65536
    nb = _TW // bn
    return pl.pallas_call(
        _combine_body,
        grid=(nb,),
        in_specs=[
            pl.BlockSpec((_BASIS_NUM, _BASIS_NUM), lambda i: (0, 0)),
            pl.BlockSpec((1, bn), lambda i: (0, i)),
            pl.BlockSpec((1, bn), lambda i: (0, i)),
            pl.BlockSpec((_BASIS_NUM - 1, bn), lambda i: (0, i)),
            pl.BlockSpec((_BASIS_NUM - 1, bn), lambda i: (0, i)),
        ],
        out_specs=pl.BlockSpec((_BASIS_NUM, bn), lambda i: (0, i)),
        out_shape=jax.ShapeDtypeStruct((_BASIS_NUM, _TW), jnp.int32),
    )(exp, me, mo, ee, eo).reshape(-1)


def _sc_body(tblf, xs_hbm, ys_hbm, zs_hbm, rays_hbm, out_hbm,
             xs, ys, zs, fr, idx_buf, gb, ob0, ob1, spm, sem0, sem1):
    cid = lax.axis_index("c")
    sid = lax.axis_index("s")
    wid = sid * 2 + cid
    lanes = jnp.arange(16, dtype=jnp.int32)
    wbase = wid * _PPW

    pltpu.sync_copy(xs_hbm.at[pl.ds(wbase, _PPW)], xs)
    pltpu.sync_copy(ys_hbm.at[pl.ds(wbase, _PPW)], ys)
    pltpu.sync_copy(zs_hbm.at[pl.ds(wbase, _PPW)], zs)
    pltpu.sync_copy(rays_hbm.at[pl.ds(wbase, _PPW)], fr)

    def fr_body(g, _):
        sl = pl.ds(g * 16, 16)
        fr[sl] = fr[sl] >> 10
        return ()
    lax.fori_loop(0, _NG, fr_body, (), unroll=False)

    def stage(stride, off8):
        cpf = stride // _CB
        w_total = _N_FRAMES * stride
        rounds = (w_total + 16 * _CB - 1) // (16 * _CB)
        bounce = gb.at[pl.ds(0, _CB)]

        def stage_round(j, _):
            k = sid + 16 * j
            coff = k * _CB

            @pl.when(coff < w_total)
            def _():
                f = k // cpf
                c = k - f * cpf
                soff = f * _TW + off8 + c * _CB
                pltpu.sync_copy(tblf.at[pl.ds(soff, _CB)], bounce)
                pltpu.sync_copy(bounce, spm.at[pl.ds(coff, _CB)])
            return ()
        lax.fori_loop(0, rounds, stage_round, (), unroll=False)
        plsc.subcore_barrier()

    def corner_setup(g, scale, grid_r1):
        sl = pl.ds(g * 16, 16)
        px = xs[sl] * scale + 0.5
        py = ys[sl] * scale + 0.5
        pz = zs[sl] * scale + 0.5
        ix = px.astype(jnp.int32)
        iy = py.astype(jnp.int32)
        iz = pz.astype(jnp.int32)
        fx = px - ix.astype(jnp.float32)
        fy = py - iy.astype(jnp.float32)
        fz = pz - iz.astype(jnp.float32)
        if grid_r1 is None:
            ya = iy * _P1
            za = iz * _P2
            mask = _HM_BIG - 1
            def cidx(cx, cy, cz):
                return (((ix + cx)
                         ^ (ya + cy * _P1)
                         ^ (za + cz * _P2)) & mask)
        else:
            r1 = grid_r1
            ya = iy * r1
            za = iz * (r1 * r1)
            def cidx(cx, cy, cz):
                return (ix + cx) + (ya + cy * r1) + (za + cz * (r1 * r1))
        return cidx, (fx, fy, fz)

    def make_pass_a(scale, grid_r1, stride, pad0, buf_words):
        def pass_a(g, _):
            cidx, _fr = corner_setup(g, scale, grid_r1)
            fbv = fr[pl.ds(g * 16, 16)] * stride + pad0
            gbase = buf_words + (g % (_CP // 16)) * 128
            for c in range(8):
                cx, cy, cz = c & 1, (c >> 1) & 1, (c >> 2) & 1
                idx_buf[pl.ds(gbase + c * 16, 16)] = cidx(cx, cy, cz) + fbv
            return ()
        return pass_a

    def make_pass_b(scale, grid_r1, buf_words):
        def pass_b(g, _):
            _cidx, (fx, fy, fz) = corner_setup(g, scale, grid_r1)
            wx0, wy0, wz0 = 1.0 - fx, 1.0 - fy, 1.0 - fz
            wxy = [wx0 * wy0, fx * wy0, wx0 * fy, fx * fy]
            gbase = buf_words + (g % (_CP // 16)) * 128
            acc0 = jnp.zeros((16,), jnp.float32)
            acc1 = jnp.zeros((16,), jnp.float32)
            for c in range(8):
                cx, cy, cz = c & 1, (c >> 1) & 1, (c >> 2) & 1
                w = wxy[cx + 2 * cy] * (fz if cz else wz0)
                vi = plsc.load_gather(gb, [gbase + c * 16 + lanes])
                v0 = plsc.bitcast(vi << 16, jnp.float32)
                v1 = plsc.bitcast(vi & jnp.int32(-65536), jnp.float32)
                acc0 = acc0 + w * v0
                acc1 = acc1 + w * v1
            sl = pl.ds(g * 16, 16)
            ob0[sl] = acc0
            ob1[sl] = acc1
            return ()
        return pass_b

    sems = (sem0, sem1)

    def run_level(scale, grid_r1, stride, pad0):
        pa = [make_pass_a(scale, grid_r1, stride, pad0, b * _CW)
              for b in (0, 1)]
        pb = [make_pass_b(scale, grid_r1, b * _CW) for b in (0, 1)]
        ngc = _CP // 16

        def start(h):
            b = h % 2
            return pltpu.async_copy(
                spm.at[idx_buf.at[pl.ds(b * _CW, _CW)]],
                gb.at[pl.ds(b * _CW, _CW)], sems[b])

        lax.fori_loop(0, ngc, pa[0], (), unroll=False)
        d0 = start(0)
        lax.fori_loop(ngc, 2 * ngc, pa[1], (), unroll=False)
        d1 = start(1)
        d0.wait()
        lax.fori_loop(0, ngc, pb[0], (), unroll=False)
        lax.fori_loop(2 * ngc, 3 * ngc, pa[0], (), unroll=False)
        d2 = start(2)
        d1.wait()
        lax.fori_loop(ngc, 2 * ngc, pb[1], (), unroll=False)
        lax.fori_loop(3 * ngc, 4 * ngc, pa[1], (), unroll=False)
        d3 = start(3)
        d2.wait()
        lax.fori_loop(2 * ngc, 3 * ngc, pb[0], (), unroll=False)
        d3.wait()
        lax.fori_loop(3 * ngc, 4 * ngc, pb[1], (), unroll=False)

    for lvl in (0, 1):
        p = _LEVEL_PARAMS[lvl]
        st = _LVL_STAGE[lvl]
        stage(st["stride"], st["off8"])
        run_level(jnp.float32(p["scale"]), p["res"] + 1, st["stride"],
                  st["pad0"])
        pltpu.sync_copy(ob0, out_hbm.at[2 * lvl, pl.ds(wbase, _PPW)])
        pltpu.sync_copy(ob1, out_hbm.at[2 * lvl + 1, pl.ds(wbase, _PPW)])
        plsc.subcore_barrier()

    def big_body(i, _):
        st = _LVL_STAGE[_BIG0]
        stage(st["stride"], st["off8"] + i * _HM_BIG)
        scale = (jnp.int32(64) << i).astype(jnp.float32) - 1.0
        run_level(scale, None, st["stride"], st["pad0"])
        lvl2 = 2 * (i + _BIG0)
        pltpu.sync_copy(ob0, out_hbm.at[lvl2, pl.ds(wbase, _PPW)])
        pltpu.sync_copy(ob1, out_hbm.at[lvl2 + 1, pl.ds(wbase, _PPW)])
        plsc.subcore_barrier()
        return ()
    lax.fori_loop(0, _NBIG, big_body, (), unroll=False)


def _sc_encode(tblf, xs, ys, zs, rays):
    mesh = plsc.VectorSubcoreMesh(core_axis_name="c", subcore_axis_name="s",
                                  num_cores=2, num_subcores=16)
    f = functools.partial(
        pl.kernel,
        out_type=jax.ShapeDtypeStruct((_OUT_D, _N_POINTS), jnp.float32),
        mesh=mesh,
        compiler_params=pltpu.CompilerParams(needs_layout_passes=False),
        scratch_types=[
            pltpu.VMEM((_PPW,), jnp.float32),
            pltpu.VMEM((_PPW,), jnp.float32),
            pltpu.VMEM((_PPW,), jnp.float32),
            pltpu.VMEM((_PPW,), jnp.int32),
            pltpu.VMEM((2 * _CW,), jnp.int32),
            pltpu.VMEM((2 * _CW,), jnp.int32),
            pltpu.VMEM((_PPW,), jnp.float32),
            pltpu.VMEM((_PPW,), jnp.float32),
            pltpu.VMEM_SHARED((_N_FRAMES * _LVL_STAGE[_BIG0]["stride"],),
                              jnp.int32),
            pltpu.SemaphoreType.DMA,
            pltpu.SemaphoreType.DMA,
        ],
    )(_sc_body)
    return f(tblf, xs, ys, zs, rays)


def kernel(inputs, exp, xyzstorays, embeddings_mean, embeddings):
    tblf = _combine_full(exp,
                         embeddings_mean[:, :, 0], embeddings_mean[:, :, 1],
                         embeddings[:, :, 0], embeddings[:, :, 1])
    xs = inputs[:, 0]
    ys = inputs[:, 1]
    zs = inputs[:, 2]
    out = _sc_encode(tblf, xs, ys, zs, xyzstorays.astype(jnp.int32))
    return out.T
